# baseline (speedup 1.0000x reference)
import functools

import jax
import jax.numpy as jnp
from jax.experimental import pallas as pl
from jax.experimental.pallas import tpu as pltpu

N = 50000
F_X = 256
H = 256
F_OUT = 128
B = 512
BN = 5000
W = 128


def _pool_body(base1_ref, noob1_ref, base2_ref, noob2_ref,
               x1_ref, b1_ref, x2_ref, b2_ref,
               wa0_ref, ba0_ref, wa1_ref, ba1_ref, wa2_ref, ba2_ref,
               su1_ref, su2_ref):
    i = pl.program_id(0)

    @pl.when(i == 0)
    def _init():
        su1_ref[...] = jnp.zeros_like(su1_ref)
        su2_ref[...] = jnp.zeros_like(su2_ref)

    wa0 = wa0_ref[...]
    wa1 = wa1_ref[...]
    wa2 = wa2_ref[...]
    ba0 = ba0_ref[...]
    ba1 = ba1_ref[...]
    ba2 = ba2_ref[0, 0]

    def one_set(x_ref, b_ref, base_ref, noob_ref, su_ref):
        xb = x_ref[...].astype(jnp.bfloat16)
        h = jnp.maximum(jnp.dot(xb, wa0, preferred_element_type=jnp.float32)
                        + ba0, 0.0).astype(jnp.bfloat16)
        h2 = jnp.maximum(jnp.dot(h, wa1, preferred_element_type=jnp.float32)
                         + ba1, 0.0)
        a = jnp.sum(h2 * wa2, axis=1, keepdims=True) + ba2
        w = xb * a.astype(jnp.bfloat16)
        ones = jnp.ones((BN, 128), jnp.bfloat16)
        we = jnp.concatenate([w, ones], axis=1)

        base = pl.multiple_of(base_ref[i], 8)
        b_row = b_ref[0]
        rel = b_row - base
        seg = jax.lax.broadcasted_iota(jnp.int32, (W, BN), 0)
        m = seg == rel
        oh = m.astype(jnp.bfloat16)
        su_ref[pl.ds(base, W), :] += jnp.dot(oh, we, preferred_element_type=jnp.float32)

        @pl.when(noob_ref[i] > 0)
        def _fallback():
            oob = rel >= W
            segf = jax.lax.broadcasted_iota(jnp.int32, (B, BN), 0)
            m2 = (segf == b_row) & oob
            oh2 = m2.astype(jnp.bfloat16)
            su_ref[...] += jnp.dot(oh2, we, preferred_element_type=jnp.float32)

    one_set(x1_ref, b1_ref, base1_ref, noob1_ref, su1_ref)
    one_set(x2_ref, b2_ref, base2_ref, noob2_ref, su2_ref)


def _final_body(su1_ref, su2_ref,
                wf0_ref, bf0_ref, wf1_ref, bf1_ref, wf2_ref, bf2_ref,
                out_ref):
    c1 = su1_ref[:, F_X:F_X + 1]
    c2 = su2_ref[:, F_X:F_X + 1]
    u1 = su1_ref[:, 0:F_X] / jnp.maximum(c1, 1.0)
    u2 = su2_ref[:, 0:F_X] / jnp.maximum(c2, 1.0)
    h = (jnp.dot(u1, wf0_ref[0:F_X, :], preferred_element_type=jnp.float32)
         + jnp.dot(u2, wf0_ref[F_X:2 * F_X, :], preferred_element_type=jnp.float32)
         + bf0_ref[...])
    h = jnp.maximum(h, 0.0)
    h = jnp.maximum(jnp.dot(h, wf1_ref[...], preferred_element_type=jnp.float32)
                    + bf1_ref[...], 0.0)
    out_ref[...] = (jnp.dot(h, wf2_ref[...], preferred_element_type=jnp.float32)
                    + bf2_ref[...])


def _block_meta(batch, g):
    blk = batch.reshape(g, BN)
    bmin = jnp.min(blk, axis=1)
    base = jnp.minimum((bmin // 8) * 8, B - W).astype(jnp.int32)
    noob = jnp.sum((blk - base[:, None]) >= W, axis=1).astype(jnp.int32)
    return base, noob


@functools.partial(jax.jit)
def kernel(x1, batch1, x2, batch2, Wa0, ba0, Wa1, ba1, Wa2, ba2,
           Wf0, bf0, Wf1, bf1, Wf2, bf2):
    g = N // BN
    b1r = batch1.reshape(g, 1, BN)
    b2r = batch2.reshape(g, 1, BN)
    base1, noob1 = _block_meta(batch1, g)
    base2, noob2 = _block_meta(batch2, g)

    full = lambda i, *_: (0, 0)
    grid_spec = pltpu.PrefetchScalarGridSpec(
        num_scalar_prefetch=4,
        grid=(g,),
        in_specs=[
            pl.BlockSpec((BN, F_X), lambda i, *_: (i, 0)),
            pl.BlockSpec((1, 1, BN), lambda i, *_: (i, 0, 0)),
            pl.BlockSpec((BN, F_X), lambda i, *_: (i, 0)),
            pl.BlockSpec((1, 1, BN), lambda i, *_: (i, 0, 0)),
            pl.BlockSpec((F_X, H), full),
            pl.BlockSpec((1, H), full),
            pl.BlockSpec((H, H), full),
            pl.BlockSpec((1, H), full),
            pl.BlockSpec((1, H), full),
            pl.BlockSpec((1, 1), full),
        ],
        out_specs=[
            pl.BlockSpec((B, F_X + 128), full),
            pl.BlockSpec((B, F_X + 128), full),
        ],
    )
    pooled = pl.pallas_call(
        _pool_body,
        grid_spec=grid_spec,
        out_shape=[
            jax.ShapeDtypeStruct((B, F_X + 128), jnp.float32),
            jax.ShapeDtypeStruct((B, F_X + 128), jnp.float32),
        ],
    )(base1, noob1, base2, noob2,
      x1, b1r, x2, b2r,
      Wa0.astype(jnp.bfloat16), ba0.reshape(1, H),
      Wa1.astype(jnp.bfloat16), ba1.reshape(1, H),
      Wa2.reshape(1, H), ba2.reshape(1, 1))
    su1, su2 = pooled

    out = pl.pallas_call(
        _final_body,
        out_shape=jax.ShapeDtypeStruct((B, F_OUT), jnp.float32),
    )(su1, su2,
      Wf0, bf0.reshape(1, H), Wf1, bf1.reshape(1, H), Wf2, bf2.reshape(1, F_OUT))
    return out


# speedup vs baseline: 1.0480x; 1.0480x over previous
import functools

import jax
import jax.numpy as jnp
from jax.experimental import pallas as pl
from jax.experimental.pallas import tpu as pltpu

N = 50000
F_X = 256
H = 256
F_OUT = 128
B = 512
BN = 5000
W = 128


def _pool_body(base1_ref, noob1_ref, base2_ref, noob2_ref,
               x1_ref, b1_ref, x2_ref, b2_ref,
               wa0_ref, ba0_ref, wa1_ref, ba1_ref, wa2_ref, ba2_ref,
               wf0_ref, bf0_ref, wf1_ref, bf1_ref, wf2_ref, bf2_ref,
               su1_ref, cnt1_ref, su2_ref, cnt2_ref, out_ref):
    i = pl.program_id(0)

    @pl.when(i == 0)
    def _init():
        su1_ref[...] = jnp.zeros_like(su1_ref)
        cnt1_ref[...] = jnp.zeros_like(cnt1_ref)
        su2_ref[...] = jnp.zeros_like(su2_ref)
        cnt2_ref[...] = jnp.zeros_like(cnt2_ref)

    wa0 = wa0_ref[...]
    wa1 = wa1_ref[...]
    wa2 = wa2_ref[...]
    ba0 = ba0_ref[...]
    ba1 = ba1_ref[...]
    ba2 = ba2_ref[0, 0]

    def one_set(x_ref, b_ref, base_ref, noob_ref, su_ref, cnt_ref):
        xb = x_ref[...].astype(jnp.bfloat16)
        h = jnp.maximum(jnp.dot(xb, wa0, preferred_element_type=jnp.float32)
                        + ba0, 0.0).astype(jnp.bfloat16)
        h2 = jnp.maximum(jnp.dot(h, wa1, preferred_element_type=jnp.float32)
                         + ba1, 0.0)
        a = jnp.sum(h2 * wa2, axis=1, keepdims=True) + ba2
        w = xb * a.astype(jnp.bfloat16)

        base = pl.multiple_of(base_ref[i], 8)
        b_row = b_ref[0]
        rel = b_row - base
        seg = jax.lax.broadcasted_iota(jnp.int32, (W, BN), 0)
        m = seg == rel
        oh = m.astype(jnp.bfloat16)
        su_ref[pl.ds(base, W), :] += jnp.dot(oh, w, preferred_element_type=jnp.float32)
        cnt_part = jnp.sum(m.astype(jnp.float32), axis=1, keepdims=True)
        cnt_ref[pl.ds(base, W), :] += jnp.broadcast_to(cnt_part, (W, 128))

        @pl.when(noob_ref[i] > 0)
        def _fallback():
            oob = rel >= W
            segf = jax.lax.broadcasted_iota(jnp.int32, (B, BN), 0)
            m2 = (segf == b_row) & oob
            oh2 = m2.astype(jnp.bfloat16)
            su_ref[...] += jnp.dot(oh2, w, preferred_element_type=jnp.float32)
            cnt2_part = jnp.sum(m2.astype(jnp.float32), axis=1, keepdims=True)
            cnt_ref[...] += jnp.broadcast_to(cnt2_part, (B, 128))

    one_set(x1_ref, b1_ref, base1_ref, noob1_ref, su1_ref, cnt1_ref)
    one_set(x2_ref, b2_ref, base2_ref, noob2_ref, su2_ref, cnt2_ref)

    @pl.when(i == pl.num_programs(0) - 1)
    def _final():
        c1 = cnt1_ref[:, 0:1]
        c2 = cnt2_ref[:, 0:1]
        u1 = su1_ref[...] / jnp.maximum(c1, 1.0)
        u2 = su2_ref[...] / jnp.maximum(c2, 1.0)
        hf = (jnp.dot(u1, wf0_ref[0:F_X, :], preferred_element_type=jnp.float32)
              + jnp.dot(u2, wf0_ref[F_X:2 * F_X, :], preferred_element_type=jnp.float32)
              + bf0_ref[...])
        hf = jnp.maximum(hf, 0.0)
        hf = jnp.maximum(jnp.dot(hf, wf1_ref[...], preferred_element_type=jnp.float32)
                         + bf1_ref[...], 0.0)
        out_ref[...] = (jnp.dot(hf, wf2_ref[...], preferred_element_type=jnp.float32)
                        + bf2_ref[...])


def _block_meta(batch, g):
    blk = batch.reshape(g, BN)
    bmin = jnp.min(blk, axis=1)
    base = jnp.minimum((bmin // 8) * 8, B - W).astype(jnp.int32)
    noob = jnp.sum((blk - base[:, None]) >= W, axis=1).astype(jnp.int32)
    return base, noob


@functools.partial(jax.jit)
def kernel(x1, batch1, x2, batch2, Wa0, ba0, Wa1, ba1, Wa2, ba2,
           Wf0, bf0, Wf1, bf1, Wf2, bf2):
    g = N // BN
    b1r = batch1.reshape(g, 1, BN)
    b2r = batch2.reshape(g, 1, BN)
    base1, noob1 = _block_meta(batch1, g)
    base2, noob2 = _block_meta(batch2, g)

    full = lambda i, *_: (0, 0)
    grid_spec = pltpu.PrefetchScalarGridSpec(
        num_scalar_prefetch=4,
        grid=(g,),
        in_specs=[
            pl.BlockSpec((BN, F_X), lambda i, *_: (i, 0)),
            pl.BlockSpec((1, 1, BN), lambda i, *_: (i, 0, 0)),
            pl.BlockSpec((BN, F_X), lambda i, *_: (i, 0)),
            pl.BlockSpec((1, 1, BN), lambda i, *_: (i, 0, 0)),
            pl.BlockSpec((F_X, H), full),
            pl.BlockSpec((1, H), full),
            pl.BlockSpec((H, H), full),
            pl.BlockSpec((1, H), full),
            pl.BlockSpec((1, H), full),
            pl.BlockSpec((1, 1), full),
            pl.BlockSpec((2 * F_X, H), full),
            pl.BlockSpec((1, H), full),
            pl.BlockSpec((H, H), full),
            pl.BlockSpec((1, H), full),
            pl.BlockSpec((H, F_OUT), full),
            pl.BlockSpec((1, F_OUT), full),
        ],
        out_specs=[
            pl.BlockSpec((B, F_X), full),
            pl.BlockSpec((B, 128), full),
            pl.BlockSpec((B, F_X), full),
            pl.BlockSpec((B, 128), full),
            pl.BlockSpec((B, F_OUT), full),
        ],
    )
    pooled = pl.pallas_call(
        _pool_body,
        grid_spec=grid_spec,
        out_shape=[
            jax.ShapeDtypeStruct((B, F_X), jnp.float32),
            jax.ShapeDtypeStruct((B, 128), jnp.float32),
            jax.ShapeDtypeStruct((B, F_X), jnp.float32),
            jax.ShapeDtypeStruct((B, 128), jnp.float32),
            jax.ShapeDtypeStruct((B, F_OUT), jnp.float32),
        ],
    )(base1, noob1, base2, noob2,
      x1, b1r, x2, b2r,
      Wa0.astype(jnp.bfloat16), ba0.reshape(1, H),
      Wa1.astype(jnp.bfloat16), ba1.reshape(1, H),
      Wa2.reshape(1, H), ba2.reshape(1, 1),
      Wf0, bf0.reshape(1, H), Wf1, bf1.reshape(1, H), Wf2, bf2.reshape(1, F_OUT))
    return pooled[4]


# speedup vs baseline: 1.0650x; 1.0162x over previous
import functools

import jax
import jax.numpy as jnp
from jax.experimental import pallas as pl
from jax.experimental.pallas import tpu as pltpu

N = 50000
F_X = 256
H = 256
F_OUT = 128
B = 512
BN = 5000
W = 128


def _pool_body(base1_ref, noob1_ref, base2_ref, noob2_ref,
               x1_ref, b1_ref, x2_ref, b2_ref,
               wa0_ref, ba0_ref, wa1_ref, ba1_ref, wa2_ref, ba2_ref,
               wf0_ref, bf0_ref, wf1_ref, bf1_ref, wf2_ref, bf2_ref,
               out_ref, su1_ref, cnt1_ref, su2_ref, cnt2_ref):
    i = pl.program_id(0)

    @pl.when(i == 0)
    def _init():
        su1_ref[...] = jnp.zeros_like(su1_ref)
        cnt1_ref[...] = jnp.zeros_like(cnt1_ref)
        su2_ref[...] = jnp.zeros_like(su2_ref)
        cnt2_ref[...] = jnp.zeros_like(cnt2_ref)

    wa0 = wa0_ref[...]
    wa1 = wa1_ref[...]
    wa2 = wa2_ref[...]
    ba0 = ba0_ref[...]
    ba1 = ba1_ref[...]
    ba2 = ba2_ref[0, 0]

    def one_set(x_ref, b_ref, base_ref, noob_ref, su_ref, cnt_ref):
        xb = x_ref[...].astype(jnp.bfloat16)
        h = jnp.maximum(jnp.dot(xb, wa0, preferred_element_type=jnp.float32)
                        + ba0, 0.0).astype(jnp.bfloat16)
        h2 = jnp.maximum(jnp.dot(h, wa1, preferred_element_type=jnp.float32)
                         + ba1, 0.0)
        a = jnp.sum(h2 * wa2, axis=1, keepdims=True) + ba2
        w = xb * a.astype(jnp.bfloat16)

        base = pl.multiple_of(base_ref[i], 8)
        b_row = b_ref[0]
        rel = b_row - base
        seg = jax.lax.broadcasted_iota(jnp.int32, (W, BN), 0)
        m = seg == rel
        oh = m.astype(jnp.bfloat16)
        su_ref[pl.ds(base, W), :] += jnp.dot(oh, w, preferred_element_type=jnp.float32)
        cnt_part = jnp.sum(m.astype(jnp.float32), axis=1, keepdims=True)
        cnt_ref[pl.ds(base, W), :] += jnp.broadcast_to(cnt_part, (W, 128))

        @pl.when(noob_ref[i] > 0)
        def _fallback():
            oob = rel >= W
            segf = jax.lax.broadcasted_iota(jnp.int32, (B, BN), 0)
            m2 = (segf == b_row) & oob
            oh2 = m2.astype(jnp.bfloat16)
            su_ref[...] += jnp.dot(oh2, w, preferred_element_type=jnp.float32)
            cnt2_part = jnp.sum(m2.astype(jnp.float32), axis=1, keepdims=True)
            cnt_ref[...] += jnp.broadcast_to(cnt2_part, (B, 128))

    one_set(x1_ref, b1_ref, base1_ref, noob1_ref, su1_ref, cnt1_ref)
    one_set(x2_ref, b2_ref, base2_ref, noob2_ref, su2_ref, cnt2_ref)

    @pl.when(i == pl.num_programs(0) - 1)
    def _final():
        c1 = cnt1_ref[:, 0:1]
        c2 = cnt2_ref[:, 0:1]
        u1 = su1_ref[...] / jnp.maximum(c1, 1.0)
        u2 = su2_ref[...] / jnp.maximum(c2, 1.0)
        hf = (jnp.dot(u1, wf0_ref[0:F_X, :], preferred_element_type=jnp.float32)
              + jnp.dot(u2, wf0_ref[F_X:2 * F_X, :], preferred_element_type=jnp.float32)
              + bf0_ref[...])
        hf = jnp.maximum(hf, 0.0)
        hf = jnp.maximum(jnp.dot(hf, wf1_ref[...], preferred_element_type=jnp.float32)
                         + bf1_ref[...], 0.0)
        out_ref[...] = (jnp.dot(hf, wf2_ref[...], preferred_element_type=jnp.float32)
                        + bf2_ref[...])


def _block_meta(batch, g):
    blk = batch.reshape(g, BN)
    bmin = jnp.min(blk, axis=1)
    base = jnp.minimum((bmin // 8) * 8, B - W).astype(jnp.int32)
    noob = jnp.sum((blk - base[:, None]) >= W, axis=1).astype(jnp.int32)
    return base, noob


@functools.partial(jax.jit)
def kernel(x1, batch1, x2, batch2, Wa0, ba0, Wa1, ba1, Wa2, ba2,
           Wf0, bf0, Wf1, bf1, Wf2, bf2):
    g = N // BN
    b1r = batch1.reshape(g, 1, BN)
    b2r = batch2.reshape(g, 1, BN)
    base1, noob1 = _block_meta(batch1, g)
    base2, noob2 = _block_meta(batch2, g)

    full = lambda i, *_: (0, 0)
    grid_spec = pltpu.PrefetchScalarGridSpec(
        num_scalar_prefetch=4,
        grid=(g,),
        in_specs=[
            pl.BlockSpec((BN, F_X), lambda i, *_: (i, 0)),
            pl.BlockSpec((1, 1, BN), lambda i, *_: (i, 0, 0)),
            pl.BlockSpec((BN, F_X), lambda i, *_: (i, 0)),
            pl.BlockSpec((1, 1, BN), lambda i, *_: (i, 0, 0)),
            pl.BlockSpec((F_X, H), full),
            pl.BlockSpec((1, H), full),
            pl.BlockSpec((H, H), full),
            pl.BlockSpec((1, H), full),
            pl.BlockSpec((1, H), full),
            pl.BlockSpec((1, 1), full),
            pl.BlockSpec((2 * F_X, H), full),
            pl.BlockSpec((1, H), full),
            pl.BlockSpec((H, H), full),
            pl.BlockSpec((1, H), full),
            pl.BlockSpec((H, F_OUT), full),
            pl.BlockSpec((1, F_OUT), full),
        ],
        out_specs=[
            pl.BlockSpec((B, F_OUT), full),
        ],
        scratch_shapes=[
            pltpu.VMEM((B, F_X), jnp.float32),
            pltpu.VMEM((B, 128), jnp.float32),
            pltpu.VMEM((B, F_X), jnp.float32),
            pltpu.VMEM((B, 128), jnp.float32),
        ],
    )
    pooled = pl.pallas_call(
        _pool_body,
        grid_spec=grid_spec,
        out_shape=[
            jax.ShapeDtypeStruct((B, F_OUT), jnp.float32),
        ],
    )(base1, noob1, base2, noob2,
      x1, b1r, x2, b2r,
      Wa0.astype(jnp.bfloat16), ba0.reshape(1, H),
      Wa1.astype(jnp.bfloat16), ba1.reshape(1, H),
      Wa2.reshape(1, H), ba2.reshape(1, 1),
      Wf0, bf0.reshape(1, H), Wf1, bf1.reshape(1, H), Wf2, bf2.reshape(1, F_OUT))
    return pooled[0]


# speedup vs baseline: 1.0779x; 1.0121x over previous
import functools

import jax
import jax.numpy as jnp
from jax.experimental import pallas as pl
from jax.experimental.pallas import tpu as pltpu

N = 50000
F_X = 256
H = 256
F_OUT = 128
B = 512
BN = 5000
W = 96


def _pool_body(base1_ref, noob1_ref, base2_ref, noob2_ref,
               x1_ref, b1_ref, x2_ref, b2_ref,
               wa0_ref, ba0_ref, wa1_ref, ba1_ref, wa2_ref, ba2_ref,
               wf0_ref, bf0_ref, wf1_ref, bf1_ref, wf2_ref, bf2_ref,
               out_ref, su1_ref, cnt1_ref, su2_ref, cnt2_ref):
    i = pl.program_id(0)

    @pl.when(i == 0)
    def _init():
        su1_ref[...] = jnp.zeros_like(su1_ref)
        cnt1_ref[...] = jnp.zeros_like(cnt1_ref)
        su2_ref[...] = jnp.zeros_like(su2_ref)
        cnt2_ref[...] = jnp.zeros_like(cnt2_ref)

    wa0 = wa0_ref[...]
    wa1 = wa1_ref[...]
    wa2 = wa2_ref[...]
    ba0 = ba0_ref[...]
    ba1 = ba1_ref[...]
    ba2 = ba2_ref[0, 0]

    def one_set(x_ref, b_ref, base_ref, noob_ref, su_ref, cnt_ref):
        xb = x_ref[...].astype(jnp.bfloat16)
        h = jnp.maximum(jnp.dot(xb, wa0, preferred_element_type=jnp.float32)
                        + ba0, 0.0).astype(jnp.bfloat16)
        h2 = jnp.maximum(jnp.dot(h, wa1, preferred_element_type=jnp.float32)
                         + ba1, 0.0)
        a = jnp.sum(h2 * wa2, axis=1, keepdims=True) + ba2
        w = xb * a.astype(jnp.bfloat16)

        base = pl.multiple_of(base_ref[i], 8)
        b_row = b_ref[0]
        rel = b_row - base
        seg = jax.lax.broadcasted_iota(jnp.int32, (W, BN), 0)
        m = seg == rel
        oh = m.astype(jnp.bfloat16)
        su_ref[pl.ds(base, W), :] += jnp.dot(oh, w, preferred_element_type=jnp.float32)
        cnt_part = jnp.sum(m.astype(jnp.float32), axis=1, keepdims=True)
        cnt_ref[pl.ds(base, W), :] += jnp.broadcast_to(cnt_part, (W, 128))

        @pl.when(noob_ref[i] > 0)
        def _fallback():
            oob = rel >= W
            segf = jax.lax.broadcasted_iota(jnp.int32, (B, BN), 0)
            m2 = (segf == b_row) & oob
            oh2 = m2.astype(jnp.bfloat16)
            su_ref[...] += jnp.dot(oh2, w, preferred_element_type=jnp.float32)
            cnt2_part = jnp.sum(m2.astype(jnp.float32), axis=1, keepdims=True)
            cnt_ref[...] += jnp.broadcast_to(cnt2_part, (B, 128))

    one_set(x1_ref, b1_ref, base1_ref, noob1_ref, su1_ref, cnt1_ref)
    one_set(x2_ref, b2_ref, base2_ref, noob2_ref, su2_ref, cnt2_ref)

    @pl.when(i == pl.num_programs(0) - 1)
    def _final():
        c1 = cnt1_ref[:, 0:1]
        c2 = cnt2_ref[:, 0:1]
        u1 = su1_ref[...] / jnp.maximum(c1, 1.0)
        u2 = su2_ref[...] / jnp.maximum(c2, 1.0)
        hf = (jnp.dot(u1, wf0_ref[0:F_X, :], preferred_element_type=jnp.float32)
              + jnp.dot(u2, wf0_ref[F_X:2 * F_X, :], preferred_element_type=jnp.float32)
              + bf0_ref[...])
        hf = jnp.maximum(hf, 0.0)
        hf = jnp.maximum(jnp.dot(hf, wf1_ref[...], preferred_element_type=jnp.float32)
                         + bf1_ref[...], 0.0)
        out_ref[...] = (jnp.dot(hf, wf2_ref[...], preferred_element_type=jnp.float32)
                        + bf2_ref[...])


def _block_meta(batch, g):
    blk = batch.reshape(g, BN)
    bmin = jnp.min(blk, axis=1)
    base = jnp.minimum((bmin // 8) * 8, B - W).astype(jnp.int32)
    noob = jnp.sum((blk - base[:, None]) >= W, axis=1).astype(jnp.int32)
    return base, noob


@functools.partial(jax.jit)
def kernel(x1, batch1, x2, batch2, Wa0, ba0, Wa1, ba1, Wa2, ba2,
           Wf0, bf0, Wf1, bf1, Wf2, bf2):
    g = N // BN
    b1r = batch1.reshape(g, 1, BN)
    b2r = batch2.reshape(g, 1, BN)
    base1, noob1 = _block_meta(batch1, g)
    base2, noob2 = _block_meta(batch2, g)

    full = lambda i, *_: (0, 0)
    grid_spec = pltpu.PrefetchScalarGridSpec(
        num_scalar_prefetch=4,
        grid=(g,),
        in_specs=[
            pl.BlockSpec((BN, F_X), lambda i, *_: (i, 0)),
            pl.BlockSpec((1, 1, BN), lambda i, *_: (i, 0, 0)),
            pl.BlockSpec((BN, F_X), lambda i, *_: (i, 0)),
            pl.BlockSpec((1, 1, BN), lambda i, *_: (i, 0, 0)),
            pl.BlockSpec((F_X, H), full),
            pl.BlockSpec((1, H), full),
            pl.BlockSpec((H, H), full),
            pl.BlockSpec((1, H), full),
            pl.BlockSpec((1, H), full),
            pl.BlockSpec((1, 1), full),
            pl.BlockSpec((2 * F_X, H), full),
            pl.BlockSpec((1, H), full),
            pl.BlockSpec((H, H), full),
            pl.BlockSpec((1, H), full),
            pl.BlockSpec((H, F_OUT), full),
            pl.BlockSpec((1, F_OUT), full),
        ],
        out_specs=[
            pl.BlockSpec((B, F_OUT), full),
        ],
        scratch_shapes=[
            pltpu.VMEM((B, F_X), jnp.float32),
            pltpu.VMEM((B, 128), jnp.float32),
            pltpu.VMEM((B, F_X), jnp.float32),
            pltpu.VMEM((B, 128), jnp.float32),
        ],
    )
    pooled = pl.pallas_call(
        _pool_body,
        grid_spec=grid_spec,
        out_shape=[
            jax.ShapeDtypeStruct((B, F_OUT), jnp.float32),
        ],
    )(base1, noob1, base2, noob2,
      x1, b1r, x2, b2r,
      Wa0.astype(jnp.bfloat16), ba0.reshape(1, H),
      Wa1.astype(jnp.bfloat16), ba1.reshape(1, H),
      Wa2.reshape(1, H), ba2.reshape(1, 1),
      Wf0, bf0.reshape(1, H), Wf1, bf1.reshape(1, H), Wf2, bf2.reshape(1, F_OUT))
    return pooled[0]


# speedup vs baseline: 1.0826x; 1.0043x over previous
import functools

import jax
import jax.numpy as jnp
from jax.experimental import pallas as pl
from jax.experimental.pallas import tpu as pltpu

N = 50000
F_X = 256
H = 256
F_OUT = 128
B = 512
BN = 5000
W = 64


def _pool_body(base1_ref, noob1_ref, base2_ref, noob2_ref,
               x1_ref, b1_ref, x2_ref, b2_ref,
               wa0_ref, ba0_ref, wa1_ref, ba1_ref, wa2_ref, ba2_ref,
               wf0_ref, bf0_ref, wf1_ref, bf1_ref, wf2_ref, bf2_ref,
               out_ref, su1_ref, cnt1_ref, su2_ref, cnt2_ref):
    i = pl.program_id(0)

    @pl.when(i == 0)
    def _init():
        su1_ref[...] = jnp.zeros_like(su1_ref)
        cnt1_ref[...] = jnp.zeros_like(cnt1_ref)
        su2_ref[...] = jnp.zeros_like(su2_ref)
        cnt2_ref[...] = jnp.zeros_like(cnt2_ref)

    wa0 = wa0_ref[...]
    wa1 = wa1_ref[...]
    wa2 = wa2_ref[...]
    ba0 = ba0_ref[...]
    ba1 = ba1_ref[...]
    ba2 = ba2_ref[0, 0]

    def one_set(x_ref, b_ref, base_ref, noob_ref, su_ref, cnt_ref):
        xb = x_ref[...].astype(jnp.bfloat16)
        h = jnp.maximum(jnp.dot(xb, wa0, preferred_element_type=jnp.float32)
                        + ba0, 0.0).astype(jnp.bfloat16)
        h2 = jnp.maximum(jnp.dot(h, wa1, preferred_element_type=jnp.float32)
                         + ba1, 0.0)
        a = jnp.sum(h2 * wa2, axis=1, keepdims=True) + ba2
        w = xb * a.astype(jnp.bfloat16)

        base = pl.multiple_of(base_ref[i], 8)
        b_row = b_ref[0]
        rel = b_row - base
        seg = jax.lax.broadcasted_iota(jnp.int32, (W, BN), 0)
        m = seg == rel
        oh = m.astype(jnp.bfloat16)
        su_ref[pl.ds(base, W), :] += jnp.dot(oh, w, preferred_element_type=jnp.float32)
        cnt_part = jnp.sum(m.astype(jnp.float32), axis=1, keepdims=True)
        cnt_ref[pl.ds(base, W), :] += jnp.broadcast_to(cnt_part, (W, 128))

        @pl.when(noob_ref[i] > 0)
        def _fallback():
            oob = rel >= W
            segf = jax.lax.broadcasted_iota(jnp.int32, (B, BN), 0)
            m2 = (segf == b_row) & oob
            oh2 = m2.astype(jnp.bfloat16)
            su_ref[...] += jnp.dot(oh2, w, preferred_element_type=jnp.float32)
            cnt2_part = jnp.sum(m2.astype(jnp.float32), axis=1, keepdims=True)
            cnt_ref[...] += jnp.broadcast_to(cnt2_part, (B, 128))

    one_set(x1_ref, b1_ref, base1_ref, noob1_ref, su1_ref, cnt1_ref)
    one_set(x2_ref, b2_ref, base2_ref, noob2_ref, su2_ref, cnt2_ref)

    @pl.when(i == pl.num_programs(0) - 1)
    def _final():
        c1 = cnt1_ref[:, 0:1]
        c2 = cnt2_ref[:, 0:1]
        u1 = su1_ref[...] / jnp.maximum(c1, 1.0)
        u2 = su2_ref[...] / jnp.maximum(c2, 1.0)
        hf = (jnp.dot(u1, wf0_ref[0:F_X, :], preferred_element_type=jnp.float32)
              + jnp.dot(u2, wf0_ref[F_X:2 * F_X, :], preferred_element_type=jnp.float32)
              + bf0_ref[...])
        hf = jnp.maximum(hf, 0.0)
        hf = jnp.maximum(jnp.dot(hf, wf1_ref[...], preferred_element_type=jnp.float32)
                         + bf1_ref[...], 0.0)
        out_ref[...] = (jnp.dot(hf, wf2_ref[...], preferred_element_type=jnp.float32)
                        + bf2_ref[...])


def _block_meta(batch, g):
    blk = batch.reshape(g, BN)
    bmin = jnp.min(blk, axis=1)
    base = jnp.minimum((bmin // 8) * 8, B - W).astype(jnp.int32)
    noob = jnp.sum((blk - base[:, None]) >= W, axis=1).astype(jnp.int32)
    return base, noob


@functools.partial(jax.jit)
def kernel(x1, batch1, x2, batch2, Wa0, ba0, Wa1, ba1, Wa2, ba2,
           Wf0, bf0, Wf1, bf1, Wf2, bf2):
    g = N // BN
    b1r = batch1.reshape(g, 1, BN)
    b2r = batch2.reshape(g, 1, BN)
    base1, noob1 = _block_meta(batch1, g)
    base2, noob2 = _block_meta(batch2, g)

    full = lambda i, *_: (0, 0)
    grid_spec = pltpu.PrefetchScalarGridSpec(
        num_scalar_prefetch=4,
        grid=(g,),
        in_specs=[
            pl.BlockSpec((BN, F_X), lambda i, *_: (i, 0)),
            pl.BlockSpec((1, 1, BN), lambda i, *_: (i, 0, 0)),
            pl.BlockSpec((BN, F_X), lambda i, *_: (i, 0)),
            pl.BlockSpec((1, 1, BN), lambda i, *_: (i, 0, 0)),
            pl.BlockSpec((F_X, H), full),
            pl.BlockSpec((1, H), full),
            pl.BlockSpec((H, H), full),
            pl.BlockSpec((1, H), full),
            pl.BlockSpec((1, H), full),
            pl.BlockSpec((1, 1), full),
            pl.BlockSpec((2 * F_X, H), full),
            pl.BlockSpec((1, H), full),
            pl.BlockSpec((H, H), full),
            pl.BlockSpec((1, H), full),
            pl.BlockSpec((H, F_OUT), full),
            pl.BlockSpec((1, F_OUT), full),
        ],
        out_specs=[
            pl.BlockSpec((B, F_OUT), full),
        ],
        scratch_shapes=[
            pltpu.VMEM((B, F_X), jnp.float32),
            pltpu.VMEM((B, 128), jnp.float32),
            pltpu.VMEM((B, F_X), jnp.float32),
            pltpu.VMEM((B, 128), jnp.float32),
        ],
    )
    pooled = pl.pallas_call(
        _pool_body,
        grid_spec=grid_spec,
        out_shape=[
            jax.ShapeDtypeStruct((B, F_OUT), jnp.float32),
        ],
    )(base1, noob1, base2, noob2,
      x1, b1r, x2, b2r,
      Wa0.astype(jnp.bfloat16), ba0.reshape(1, H),
      Wa1.astype(jnp.bfloat16), ba1.reshape(1, H),
      Wa2.reshape(1, H), ba2.reshape(1, 1),
      Wf0, bf0.reshape(1, H), Wf1, bf1.reshape(1, H), Wf2, bf2.reshape(1, F_OUT))
    return pooled[0]
